# baseline (speedup 1.0000x reference)
import functools

import jax
import jax.numpy as jnp
import numpy as np
from jax import lax
from jax.experimental import pallas as pl
from jax.experimental.pallas import tpu as pltpu
from jax.experimental.pallas import tpu_sc as plsc

_N_KF = 512
_F = 1024
_K = 20
_NROWS = 2 * _K + 2
_PREC = jax.lax.Precision.HIGHEST


def _select_kernel(locs_t_ref, last_ref, idx_ref, locs_out_ref):
    diff = locs_t_ref[...] - last_ref[...]
    d = jnp.sqrt(jnp.sum(diff * diff, axis=0, keepdims=True))
    dcol = d.reshape(_N_KF, 1)
    irow = jax.lax.broadcasted_iota(jnp.int32, (1, _N_KF), 1)
    icol = jax.lax.broadcasted_iota(jnp.int32, (_N_KF, 1), 0)
    smaller = (d < dcol) | ((d == dcol) & (irow < icol))
    rank = jnp.sum(smaller.astype(jnp.int32), axis=1, keepdims=True)
    j64 = jax.lax.broadcasted_iota(jnp.int32, (1, 64), 1)
    onehot = (rank == (j64 - _K)).astype(jnp.float32)
    idxf = jax.lax.dot_general(
        irow.astype(jnp.float32), onehot,
        (((1,), (0,)), ((), ())), precision=_PREC)
    idx = jnp.where(j64 < _K, (_N_KF - _K) + j64,
                    idxf.astype(jnp.int32))
    idx_ref[...] = idx
    sel = (icol == idx).astype(jnp.float32)
    locs_out_ref[...] = jax.lax.dot_general(
        locs_t_ref[...], sel, (((1,), (0,)), ((), ())), precision=_PREC)


def _reg_gather(vec, idx):
    dnums = lax.GatherDimensionNumbers(
        offset_dims=(), collapsed_slice_dims=(0,), start_index_map=(0,))
    return lax.gather(vec, idx[:, None], dnums, (1,),
                      mode=lax.GatherScatterMode.PROMISE_IN_BOUNDS)


def _sc_row(r, src_flat, rowbuf, kpts_out, out_r):
    row_words = 3 * _F
    pltpu.sync_copy(src_flat.at[pl.ds(r * row_words, row_words)], rowbuf)
    pltpu.sync_copy(rowbuf, kpts_out.at[pl.ds(out_r * row_words, row_words)])


def _sc_kernel(kf_kpts_flat, nf_kpts_flat, idx_hbm, kpts_out, idxall, rowbuf):
    wid = lax.axis_index("s") * 2 + lax.axis_index("c")
    pltpu.sync_copy(idx_hbm, idxall)

    def idx_at(j):
        chunk = idxall[pl.ds(j, 16)]
        return chunk[0].astype(jnp.int32)

    _sc_row(idx_at(wid), kf_kpts_flat, rowbuf, kpts_out, wid)

    @pl.when(wid < 8)
    def _():
        _sc_row(idx_at(wid + 32), kf_kpts_flat, rowbuf, kpts_out, wid + 32)

    @pl.when((wid >= 8) & (wid < 10))
    def _():
        _sc_row(wid - 8, nf_kpts_flat, rowbuf, kpts_out, wid + 32)


def _desc_kernel(kf_i_ref, nf_i_ref, kfd_ref, nfd_ref, kfk_ref, nfk_ref,
                 descs_out, mask_out, kscr):
    row = pl.program_id(0)

    @pl.when(row < 2 * _K)
    def _():
        descs_out[...] = kfd_ref[...]
        kscr[...] = kfk_ref[0]

    @pl.when(row >= 2 * _K)
    def _():
        descs_out[...] = nfd_ref[...]
        kscr[...] = nfk_ref[0]

    descs = descs_out[0]
    dsq = descs * descs
    ones = jnp.ones((1, descs.shape[1]), jnp.float32)
    nd2 = jax.lax.dot_general(
        ones, dsq, (((1,), (1,)), ((), ())), precision=_PREC)
    kpts = kscr[...]
    nk2 = jnp.sum(kpts * kpts, axis=0, keepdims=True)
    m = (jnp.isfinite(nk2) & jnp.isfinite(nd2)
         & (jnp.sqrt(nk2) >= 1e-6) & (jnp.sqrt(nd2) >= 1e-6))
    mask_out[0] = m.astype(jnp.int32)


def kernel(keyframe_locs, keyframe_kpts, keyframe_descs, last_loc,
           newframe_kpts, newframe_descs):
    n_kf, f = keyframe_kpts.shape[0], keyframe_kpts.shape[1]
    d = keyframe_descs.shape[2]
    b = newframe_kpts.shape[0]

    locs_t = keyframe_locs.T
    last_c = last_loc.reshape(3, 1)
    idx64, locs_sel = pl.pallas_call(
        _select_kernel,
        out_shape=[jax.ShapeDtypeStruct((1, 64), jnp.int32),
                   jax.ShapeDtypeStruct((3, 64), jnp.float32)],
    )(locs_t, last_c)
    kf_locs = locs_sel[:, :2 * _K].T

    src40 = idx64.reshape(64)[:2 * _K]
    kf_i = jnp.concatenate([src40, src40[-1:], src40[-1:]])
    nf_i = jnp.concatenate([jnp.zeros((2 * _K,), jnp.int32),
                            jnp.arange(b, dtype=jnp.int32)])
    idx128 = jnp.concatenate([src40, jnp.zeros((88,), jnp.int32)]
                             ).astype(jnp.float32)

    mesh = plsc.VectorSubcoreMesh(core_axis_name="c", subcore_axis_name="s")
    sc_call = functools.partial(
        pl.kernel, _sc_kernel, mesh=mesh,
        out_type=[jax.ShapeDtypeStruct((_NROWS * f * 3,), jnp.float32)],
        scratch_types=[
            pltpu.VMEM((128,), jnp.float32),
            pltpu.VMEM((3 * f,), jnp.float32),
        ],
    )()
    (kpts_flat,) = sc_call(
        keyframe_kpts.reshape(-1), newframe_kpts.reshape(-1), idx128)

    kf_kpts_t = jnp.swapaxes(keyframe_kpts, 1, 2)
    nf_kpts_t = jnp.swapaxes(newframe_kpts, 1, 2)

    grid_spec = pltpu.PrefetchScalarGridSpec(
        num_scalar_prefetch=2,
        grid=(_NROWS,),
        in_specs=[
            pl.BlockSpec((1, f, d), lambda i, kf, nf: (kf[i], 0, 0)),
            pl.BlockSpec((1, f, d), lambda i, kf, nf: (nf[i], 0, 0)),
            pl.BlockSpec((1, 3, f), lambda i, kf, nf: (kf[i], 0, 0)),
            pl.BlockSpec((1, 3, f), lambda i, kf, nf: (nf[i], 0, 0)),
        ],
        out_specs=[
            pl.BlockSpec((1, f, d), lambda i, kf, nf: (i, 0, 0)),
            pl.BlockSpec((1, 1, f), lambda i, kf, nf: (i, 0, 0)),
        ],
        scratch_shapes=[pltpu.VMEM((3, f), jnp.float32)],
    )
    descs, mask_i = pl.pallas_call(
        _desc_kernel,
        grid_spec=grid_spec,
        out_shape=[
            jax.ShapeDtypeStruct((_NROWS, f, d), jnp.float32),
            jax.ShapeDtypeStruct((_NROWS, 1, f), jnp.int32),
        ],
    )(kf_i, nf_i, keyframe_descs, newframe_descs, kf_kpts_t, nf_kpts_t)

    curr_kpts = kpts_flat.reshape(_NROWS, f, 3)
    curr_mask = mask_i.reshape(_NROWS, f).astype(bool)
    kf_locs = kf_locs.astype(jnp.float32)
    return (curr_kpts, descs, curr_mask, kf_locs)


# speedup vs baseline: 1.5766x; 1.5766x over previous
import functools

import jax
import jax.numpy as jnp
import numpy as np
from jax import lax
from jax.experimental import pallas as pl
from jax.experimental.pallas import tpu as pltpu
from jax.experimental.pallas import tpu_sc as plsc

_N_KF = 512
_F = 1024
_K = 20
_NROWS = 2 * _K + 2
_PREC = jax.lax.Precision.HIGHEST


def _select_kernel(locs_t_ref, last_ref, idx_ref, locs_out_ref):
    diff = locs_t_ref[...] - last_ref[...]
    d = jnp.sqrt(jnp.sum(diff * diff, axis=0, keepdims=True))
    dcol = d.reshape(_N_KF, 1)
    irow = jax.lax.broadcasted_iota(jnp.int32, (1, _N_KF), 1)
    icol = jax.lax.broadcasted_iota(jnp.int32, (_N_KF, 1), 0)
    smaller = (d < dcol) | ((d == dcol) & (irow < icol))
    rank = jnp.sum(smaller.astype(jnp.int32), axis=1, keepdims=True)
    j64 = jax.lax.broadcasted_iota(jnp.int32, (1, 64), 1)
    onehot = (rank == (j64 - _K)).astype(jnp.float32)
    idxf = jax.lax.dot_general(
        irow.astype(jnp.float32), onehot,
        (((1,), (0,)), ((), ())), precision=_PREC)
    idx = jnp.where(j64 < _K, (_N_KF - _K) + j64,
                    idxf.astype(jnp.int32))
    idx_ref[...] = idx
    sel = (icol == idx).astype(jnp.float32)
    locs_out_ref[...] = jax.lax.dot_general(
        locs_t_ref[...], sel, (((1,), (0,)), ((), ())), precision=_PREC)


def _reg_gather(vec, idx):
    dnums = lax.GatherDimensionNumbers(
        offset_dims=(), collapsed_slice_dims=(0,), start_index_map=(0,))
    return lax.gather(vec, idx[:, None], dnums, (1,),
                      mode=lax.GatherScatterMode.PROMISE_IN_BOUNDS)


def _sc_row(r, src, kpts_out, out_r):
    pltpu.sync_copy(src.at[r], kpts_out.at[out_r])


def _sc_kernel(kf_kpts, nf_kpts, idx_hbm, kpts_out, idxall):
    wid = lax.axis_index("s") * 2 + lax.axis_index("c")
    pltpu.sync_copy(idx_hbm, idxall)

    def idx_at(j):
        chunk = idxall[pl.ds(j, 16)]
        return chunk[0].astype(jnp.int32)

    _sc_row(idx_at(wid), kf_kpts, kpts_out, wid)

    @pl.when(wid < 8)
    def _():
        _sc_row(idx_at(wid + 32), kf_kpts, kpts_out, wid + 32)

    @pl.when((wid >= 8) & (wid < 10))
    def _():
        _sc_row(wid - 8, nf_kpts, kpts_out, wid + 32)


def _desc_kernel(kf_i_ref, nf_i_ref, kfd_ref, nfd_ref, kfk_ref, nfk_ref,
                 descs_out, mask_out, kscr):
    row = pl.program_id(0)

    @pl.when(row < 2 * _K)
    def _():
        descs_out[...] = kfd_ref[...]
        kscr[...] = kfk_ref[0]

    @pl.when(row >= 2 * _K)
    def _():
        descs_out[...] = nfd_ref[...]
        kscr[...] = nfk_ref[0]

    descs = descs_out[0]
    dsq = descs * descs
    ones = jnp.ones((1, descs.shape[1]), jnp.float32)
    nd2 = jax.lax.dot_general(
        ones, dsq, (((1,), (1,)), ((), ())), precision=_PREC)
    kpts = kscr[...]
    nk2 = jnp.sum(kpts * kpts, axis=0, keepdims=True)
    m = (jnp.isfinite(nk2) & jnp.isfinite(nd2)
         & (jnp.sqrt(nk2) >= 1e-6) & (jnp.sqrt(nd2) >= 1e-6))
    mask_out[0] = m.astype(jnp.int32)


def kernel(keyframe_locs, keyframe_kpts, keyframe_descs, last_loc,
           newframe_kpts, newframe_descs):
    n_kf, f = keyframe_kpts.shape[0], keyframe_kpts.shape[1]
    d = keyframe_descs.shape[2]
    b = newframe_kpts.shape[0]

    locs_t = keyframe_locs.T
    last_c = last_loc.reshape(3, 1)
    idx64, locs_sel = pl.pallas_call(
        _select_kernel,
        out_shape=[jax.ShapeDtypeStruct((1, 64), jnp.int32),
                   jax.ShapeDtypeStruct((3, 64), jnp.float32)],
    )(locs_t, last_c)
    kf_locs = locs_sel[:, :2 * _K].T

    src40 = idx64.reshape(64)[:2 * _K]
    kf_i = jnp.concatenate([src40, src40[-1:], src40[-1:]])
    nf_i = jnp.concatenate([jnp.zeros((2 * _K,), jnp.int32),
                            jnp.arange(b, dtype=jnp.int32)])
    idx128 = jnp.concatenate([src40, jnp.zeros((88,), jnp.int32)]
                             ).astype(jnp.float32)

    mesh = plsc.VectorSubcoreMesh(core_axis_name="c", subcore_axis_name="s")
    sc_call = functools.partial(
        pl.kernel, _sc_kernel, mesh=mesh,
        out_type=[jax.ShapeDtypeStruct((_NROWS, f, 3), jnp.float32)],
        scratch_types=[
            pltpu.VMEM((128,), jnp.float32),
        ],
    )()
    (curr_kpts,) = sc_call(keyframe_kpts, newframe_kpts, idx128)

    kf_kpts_t = jnp.swapaxes(keyframe_kpts, 1, 2)
    nf_kpts_t = jnp.swapaxes(newframe_kpts, 1, 2)

    grid_spec = pltpu.PrefetchScalarGridSpec(
        num_scalar_prefetch=2,
        grid=(_NROWS,),
        in_specs=[
            pl.BlockSpec((1, f, d), lambda i, kf, nf: (kf[i], 0, 0)),
            pl.BlockSpec((1, f, d), lambda i, kf, nf: (nf[i], 0, 0)),
            pl.BlockSpec((1, 3, f), lambda i, kf, nf: (kf[i], 0, 0)),
            pl.BlockSpec((1, 3, f), lambda i, kf, nf: (nf[i], 0, 0)),
        ],
        out_specs=[
            pl.BlockSpec((1, f, d), lambda i, kf, nf: (i, 0, 0)),
            pl.BlockSpec((1, 1, f), lambda i, kf, nf: (i, 0, 0)),
        ],
        scratch_shapes=[pltpu.VMEM((3, f), jnp.float32)],
    )
    descs, mask_i = pl.pallas_call(
        _desc_kernel,
        grid_spec=grid_spec,
        out_shape=[
            jax.ShapeDtypeStruct((_NROWS, f, d), jnp.float32),
            jax.ShapeDtypeStruct((_NROWS, 1, f), jnp.int32),
        ],
    )(kf_i, nf_i, keyframe_descs, newframe_descs, kf_kpts_t, nf_kpts_t)

    curr_mask = mask_i.reshape(_NROWS, f).astype(bool)
    kf_locs = kf_locs.astype(jnp.float32)
    return (curr_kpts, descs, curr_mask, kf_locs)


# speedup vs baseline: 7.0236x; 4.4550x over previous
import functools

import jax
import jax.numpy as jnp
from jax.experimental import pallas as pl
from jax.experimental.pallas import tpu as pltpu

_N_KF = 512
_K = 20
_NROWS = 2 * _K + 2
_PREC = jax.lax.Precision.HIGHEST


def _select_kernel(locs_t_ref, last_ref, idx_ref, locs_out_ref):
    diff = locs_t_ref[...] - last_ref[...]
    d = jnp.sqrt(jnp.sum(diff * diff, axis=0, keepdims=True))
    dcol = d.reshape(_N_KF, 1)
    irow = jax.lax.broadcasted_iota(jnp.int32, (1, _N_KF), 1)
    icol = jax.lax.broadcasted_iota(jnp.int32, (_N_KF, 1), 0)
    smaller = (d < dcol) | ((d == dcol) & (irow < icol))
    rank = jnp.sum(smaller.astype(jnp.int32), axis=1, keepdims=True)
    j64 = jax.lax.broadcasted_iota(jnp.int32, (1, 64), 1)
    onehot = (rank == (j64 - _K)).astype(jnp.float32)
    idxf = jax.lax.dot_general(
        irow.astype(jnp.float32), onehot,
        (((1,), (0,)), ((), ())), precision=_PREC)
    idx = jnp.where(j64 < _K, (_N_KF - _K) + j64,
                    idxf.astype(jnp.int32))
    idx_ref[...] = idx
    sel = (icol == idx).astype(jnp.float32)
    locs_out_ref[...] = jax.lax.dot_general(
        locs_t_ref[...], sel, (((1,), (0,)), ((), ())), precision=_PREC)


def _gather_kernel(kf_i_ref, nf_i_ref, kfk_ref, kfd_ref,
                   nfk_ref, nfd_ref, kpts_out, descs_out, mask_out):
    row = pl.program_id(0)

    @pl.when(row < 2 * _K)
    def _():
        kpts_out[0] = jnp.transpose(kfk_ref[0])
        descs_out[...] = kfd_ref[...]

    @pl.when(row >= 2 * _K)
    def _():
        kpts_out[0] = jnp.transpose(nfk_ref[0])
        descs_out[...] = nfd_ref[...]

    kpts = kpts_out[0]
    descs = descs_out[0]
    nk2 = jnp.sum(kpts * kpts, axis=0, keepdims=True)
    dsq = descs * descs
    ones = jnp.ones((1, descs.shape[1]), jnp.float32)
    nd2 = jax.lax.dot_general(
        ones, dsq, (((1,), (1,)), ((), ())), precision=_PREC)
    m = (jnp.isfinite(nk2) & jnp.isfinite(nd2)
         & (jnp.sqrt(nk2) >= 1e-6) & (jnp.sqrt(nd2) >= 1e-6))
    mask_out[0] = m.astype(jnp.int32)


def kernel(keyframe_locs, keyframe_kpts, keyframe_descs, last_loc,
           newframe_kpts, newframe_descs):
    n_kf, f = keyframe_kpts.shape[0], keyframe_kpts.shape[1]
    d = keyframe_descs.shape[2]
    b = newframe_kpts.shape[0]

    locs_t = keyframe_locs.T
    last_c = last_loc.reshape(3, 1)
    idx64, locs_sel = pl.pallas_call(
        _select_kernel,
        out_shape=[jax.ShapeDtypeStruct((1, 64), jnp.int32),
                   jax.ShapeDtypeStruct((3, 64), jnp.float32)],
    )(locs_t, last_c)
    kf_locs = locs_sel[:, :2 * _K].T

    src40 = idx64.reshape(64)[:2 * _K]
    kf_i = jnp.concatenate([src40, src40[-1:], src40[-1:]])
    nf_i = jnp.concatenate([jnp.zeros((2 * _K,), jnp.int32),
                            jnp.arange(b, dtype=jnp.int32)])

    grid_spec = pltpu.PrefetchScalarGridSpec(
        num_scalar_prefetch=2,
        grid=(_NROWS,),
        in_specs=[
            pl.BlockSpec((1, f, 3), lambda i, kf, nf: (kf[i], 0, 0)),
            pl.BlockSpec((1, f, d), lambda i, kf, nf: (kf[i], 0, 0)),
            pl.BlockSpec((1, f, 3), lambda i, kf, nf: (nf[i], 0, 0)),
            pl.BlockSpec((1, f, d), lambda i, kf, nf: (nf[i], 0, 0)),
        ],
        out_specs=[
            pl.BlockSpec((1, 3, f), lambda i, kf, nf: (i, 0, 0)),
            pl.BlockSpec((1, f, d), lambda i, kf, nf: (i, 0, 0)),
            pl.BlockSpec((1, 1, f), lambda i, kf, nf: (i, 0, 0)),
        ],
    )
    kpts_t, descs, mask_i = pl.pallas_call(
        _gather_kernel,
        grid_spec=grid_spec,
        out_shape=[
            jax.ShapeDtypeStruct((_NROWS, 3, f), jnp.float32),
            jax.ShapeDtypeStruct((_NROWS, f, d), jnp.float32),
            jax.ShapeDtypeStruct((_NROWS, 1, f), jnp.int32),
        ],
    )(kf_i, nf_i, keyframe_kpts, keyframe_descs, newframe_kpts,
      newframe_descs)

    curr_kpts = jnp.swapaxes(kpts_t, 1, 2)
    curr_mask = mask_i.reshape(_NROWS, f).astype(bool)
    return (curr_kpts, descs, curr_mask, kf_locs)


# speedup vs baseline: 18.7331x; 2.6672x over previous
import functools

import jax
import jax.numpy as jnp
from jax.experimental import pallas as pl
from jax.experimental.pallas import tpu as pltpu

_N_KF = 512
_K = 20
_NROWS = 2 * _K + 2
_PREC = jax.lax.Precision.HIGHEST


def _select_kernel(locs_t_ref, last_ref, idx_ref, locs_out_ref):
    diff = locs_t_ref[...] - last_ref[...]
    d = jnp.sqrt(jnp.sum(diff * diff, axis=0, keepdims=True))
    dcol = d.reshape(_N_KF, 1)
    irow = jax.lax.broadcasted_iota(jnp.int32, (1, _N_KF), 1)
    icol = jax.lax.broadcasted_iota(jnp.int32, (_N_KF, 1), 0)
    smaller = (d < dcol) | ((d == dcol) & (irow < icol))
    rank = jnp.sum(smaller.astype(jnp.int32), axis=1, keepdims=True)
    j64 = jax.lax.broadcasted_iota(jnp.int32, (1, 64), 1)
    onehot = (rank == (j64 - _K)).astype(jnp.float32)
    idxf = jax.lax.dot_general(
        irow.astype(jnp.float32), onehot,
        (((1,), (0,)), ((), ())), precision=_PREC)
    idx = jnp.where(j64 < _K, (_N_KF - _K) + j64,
                    idxf.astype(jnp.int32))
    idx_ref[...] = idx
    sel = (icol == idx).astype(jnp.float32)
    locs_out_ref[...] = jax.lax.dot_general(
        locs_t_ref[...], sel, (((1,), (0,)), ((), ())), precision=_PREC)


def _gather_kernel(kf_i_ref, nf_i_ref, kfk_ref, kfd_ref,
                   nfk_ref, nfd_ref, kpts_out, descs_out, mask_out):
    row = pl.program_id(0)

    @pl.when(row < 2 * _K)
    def _():
        kpts_out[...] = kfk_ref[...]
        descs_out[...] = kfd_ref[...]

    @pl.when(row >= 2 * _K)
    def _():
        kpts_out[...] = nfk_ref[...]
        descs_out[...] = nfd_ref[...]

    kpts = kpts_out[0]
    descs = descs_out[0]
    nk2 = jnp.sum(kpts * kpts, axis=0, keepdims=True)
    dsq = descs * descs
    ones = jnp.ones((1, descs.shape[1]), jnp.float32)
    nd2 = jax.lax.dot_general(
        ones, dsq, (((1,), (1,)), ((), ())), precision=_PREC)
    m = (jnp.isfinite(nk2) & jnp.isfinite(nd2)
         & (jnp.sqrt(nk2) >= 1e-6) & (jnp.sqrt(nd2) >= 1e-6))
    mask_out[0] = m.astype(jnp.int32)


def kernel(keyframe_locs, keyframe_kpts, keyframe_descs, last_loc,
           newframe_kpts, newframe_descs):
    n_kf, f = keyframe_kpts.shape[0], keyframe_kpts.shape[1]
    d = keyframe_descs.shape[2]
    b = newframe_kpts.shape[0]

    locs_t = keyframe_locs.T
    last_c = last_loc.reshape(3, 1)
    idx64, locs_sel = pl.pallas_call(
        _select_kernel,
        out_shape=[jax.ShapeDtypeStruct((1, 64), jnp.int32),
                   jax.ShapeDtypeStruct((3, 64), jnp.float32)],
    )(locs_t, last_c)
    kf_locs = locs_sel[:, :2 * _K].T

    src40 = idx64.reshape(64)[:2 * _K]
    kf_i = jnp.concatenate([src40, src40[-1:], src40[-1:]])
    nf_i = jnp.concatenate([jnp.zeros((2 * _K,), jnp.int32),
                            jnp.arange(b, dtype=jnp.int32)])

    kf_kpts_t = jnp.swapaxes(keyframe_kpts, 1, 2)
    nf_kpts_t = jnp.swapaxes(newframe_kpts, 1, 2)

    grid_spec = pltpu.PrefetchScalarGridSpec(
        num_scalar_prefetch=2,
        grid=(_NROWS,),
        in_specs=[
            pl.BlockSpec((1, 3, f), lambda i, kf, nf: (kf[i], 0, 0)),
            pl.BlockSpec((1, f, d), lambda i, kf, nf: (kf[i], 0, 0)),
            pl.BlockSpec((1, 3, f), lambda i, kf, nf: (nf[i], 0, 0)),
            pl.BlockSpec((1, f, d), lambda i, kf, nf: (nf[i], 0, 0)),
        ],
        out_specs=[
            pl.BlockSpec((1, 3, f), lambda i, kf, nf: (i, 0, 0)),
            pl.BlockSpec((1, f, d), lambda i, kf, nf: (i, 0, 0)),
            pl.BlockSpec((1, 1, f), lambda i, kf, nf: (i, 0, 0)),
        ],
    )
    kpts_t, descs, mask_i = pl.pallas_call(
        _gather_kernel,
        grid_spec=grid_spec,
        out_shape=[
            jax.ShapeDtypeStruct((_NROWS, 3, f), jnp.float32),
            jax.ShapeDtypeStruct((_NROWS, f, d), jnp.float32),
            jax.ShapeDtypeStruct((_NROWS, 1, f), jnp.int32),
        ],
    )(kf_i, nf_i, kf_kpts_t, keyframe_descs, nf_kpts_t, newframe_descs)

    curr_kpts = jnp.swapaxes(kpts_t, 1, 2)
    curr_mask = mask_i.reshape(_NROWS, f).astype(bool)
    return (curr_kpts, descs, curr_mask, kf_locs)
